# baseline (speedup 1.0000x reference)
import numpy as np
import jax
import jax.numpy as jnp
from jax.experimental import pallas as pl
from jax.experimental.pallas import tpu as pltpu

DIM = 96
RANK = 16
E = 4
PS = 8
H = 384
W = 384
TH = 8
HALO = 8
NSTRIP = H // TH
ER = E * RANK

_m = np.arange(PS)
_ang = 2.0 * np.pi * np.outer(_m, _m) / PS
_C = np.cos(_ang)
_S = -np.sin(_ang)
_CR = (np.kron(_C, _C) - np.kron(_S, _S)).astype(np.float32)
_CI = (np.kron(_C, _S) + np.kron(_S, _C)).astype(np.float32)
_FWD = np.concatenate([_CR, _CI], axis=1)
_INV = np.concatenate([_CR, _CI], axis=0)


def _shift_cat(t, pad):
    tp = jnp.pad(t, ((0, 0), (0, 0), (pad, pad)))
    return jnp.concatenate([tp[:, :, dx:dx + W] for dx in range(2 * pad + 1)],
                           axis=0)


def _strip_kernel(xc_ref, xa_ref, xb_ref, sh_ref,
                  rw_ref, rb_ref,
                  p0_ref, p1_ref,
                  m3_ref, qdb_ref, m7_ref, kvdb_ref,
                  lnw_ref, lnb_ref, pow_ref, pob_ref,
                  w2_ref, ow_ref, ob_ref, cr_ref, ci_ref,
                  out_ref):
    N = TH * W
    xc = xc_ref[0]
    X = jnp.concatenate([xa_ref[0], xc, xb_ref[0]], axis=1)
    Xs = X[:, HALO - 3:HALO + TH + 3]
    xflat = xc.reshape(DIM, N)
    CRm = cr_ref[...]
    CIm = ci_ref[...]

    logits = jnp.dot(rw_ref[...], xflat,
                     preferred_element_type=jnp.float32) + rb_ref[...]
    lm = jnp.max(logits, axis=0, keepdims=True)
    ex_ = jnp.exp(logits - lm)
    probs = ex_ / jnp.sum(ex_, axis=0, keepdims=True)
    mx = jnp.max(probs, axis=0)
    taken = jnp.zeros((N,), jnp.bool_)
    gates = []
    for i in range(E):
        hit = jnp.logical_and(probs[i] == mx, jnp.logical_not(taken))
        gates.append(jnp.where(hit, mx, 0.0))
        taken = jnp.logical_or(taken, hit)

    h_all = jnp.dot(p0_ref[...], Xs.reshape(DIM, (TH + 6) * W),
                    preferred_element_type=jnp.float32)
    h_all = h_all.reshape(ER, TH + 6, W)
    g_all = jnp.dot(p1_ref[...], sh_ref[0].reshape(DIM, N),
                    preferred_element_type=jnp.float32)
    g_all = g_all * jax.nn.sigmoid(g_all)

    nh, nw = TH // PS, W // PS
    B = ER * nh * nw

    def to_patch(t):
        t = t.reshape(ER, nh, PS, nw, PS).transpose(0, 1, 3, 2, 4)
        return t.reshape(B, PS * PS)

    def from_patch(t):
        t = t.reshape(ER, nh, nw, PS, PS).transpose(0, 1, 3, 2, 4)
        return t.reshape(ER, TH, W)

    qpieces, kpieces, vpieces = [], [], []
    for i in range(E):
        h = h_all[RANK * i:RANK * (i + 1)]
        H3 = _shift_cat(h[:, 2:TH + 4], 1)
        qq = qdb_ref[i].reshape(RANK, 1)
        for dy in range(3):
            qq = qq + jnp.dot(m3_ref[i, dy],
                              H3[:, dy:dy + TH].reshape(3 * RANK, N),
                              preferred_element_type=jnp.float32)
        H7 = _shift_cat(h, 3)
        kv = kvdb_ref[i].reshape(2 * RANK, 1)
        for dy in range(7):
            kv = kv + jnp.dot(m7_ref[i, dy],
                              H7[:, dy:dy + TH].reshape(7 * RANK, N),
                              preferred_element_type=jnp.float32)
        qpieces.append(qq.reshape(RANK, TH, W))
        kpieces.append(kv[:RANK].reshape(RANK, TH, W))
        vpieces.append(kv[RANK:].reshape(RANK, TH, W))

    qcat = jnp.concatenate(qpieces, axis=0)
    kcat = jnp.concatenate(kpieces, axis=0)
    vcat = jnp.concatenate(vpieces, axis=0)
    QK = jnp.concatenate([to_patch(qcat), to_patch(kcat)], axis=0)
    F = jnp.dot(QK, CRm, preferred_element_type=jnp.float32)
    Aq, Bq = F[:B, :64], F[:B, 64:]
    Ak, Bk = F[B:, :64], F[B:, 64:]
    ReIm = jnp.concatenate([Aq * Ak - Bq * Bk, Aq * Bk + Bq * Ak], axis=1)
    o = jnp.dot(ReIm, CIm, preferred_element_type=jnp.float32)
    o = from_patch(o * (1.0 / 64.0))
    og = o.reshape(E, RANK, TH, W)
    mu = jnp.mean(og, axis=1, keepdims=True)
    var = jnp.mean((og - mu) ** 2, axis=1, keepdims=True)
    og = (og - mu) * jax.lax.rsqrt(var + 1e-5)
    og = og * lnw_ref[...] + lnb_ref[...]
    o = og.reshape(ER, TH, W) * vcat
    a = jnp.dot(pow_ref[...], o.reshape(ER, N),
                preferred_element_type=jnp.float32) + pob_ref[...]
    grep = jnp.broadcast_to(
        jnp.stack(gates)[:, None, :], (E, RANK, N)).reshape(ER, N)
    U = a * g_all * grep
    y = jnp.dot(w2_ref[...], U, preferred_element_type=jnp.float32)
    y = y + jnp.dot(ow_ref[...], xflat * mx[None, :],
                    preferred_element_type=jnp.float32)
    y = y + ob_ref[...]
    out_ref[...] = y.reshape(1, DIM, TH, W)


def kernel(x, shared, router_w, router_b, e_p0, e_p1, e_p2, e_qw, e_qdw,
           e_qdb, e_kvw, e_kvdw, e_kvdb, e_lnw, e_lnb, e_pow, e_pob,
           projout_w, projout_b):
    xpad = jnp.pad(x, ((0, 0), (0, 0), (TH, TH), (0, 0)))
    m3 = jnp.einsum('eoyx,eoc->eyoxc', e_qdw[:, :, 0], e_qw)
    m3 = m3.reshape(E, 3, RANK, 3 * RANK)
    m7 = jnp.einsum('eoyx,eoc->eyoxc', e_kvdw[:, :, 0], e_kvw)
    m7 = m7.reshape(E, 7, 2 * RANK, 7 * RANK)
    p0cat = e_p0.reshape(ER, DIM)
    p1cat = e_p1.reshape(ER, DIM)
    w2 = projout_w @ jnp.moveaxis(e_p2, 0, 1).reshape(DIM, ER)
    powbd = jnp.einsum('eoc,ef->eofc', e_pow,
                       jnp.eye(E, dtype=jnp.float32)).reshape(ER, ER)
    pob = e_pob.reshape(ER, 1)
    qdb = e_qdb.reshape(E, RANK, 1, 1)
    kvdb = e_kvdb.reshape(E, 2 * RANK, 1, 1)
    lnw = e_lnw.reshape(E, RANK, 1, 1)
    lnb = e_lnb.reshape(E, RANK, 1, 1)
    rb = router_b.reshape(E, 1)
    ob = projout_b.reshape(DIM, 1)

    full = lambda s: pl.BlockSpec(s, lambda i: (0,) * len(s))
    grid_spec = pl.GridSpec(
        grid=(NSTRIP,),
        in_specs=[
            pl.BlockSpec((1, DIM, TH, W), lambda i: (0, 0, i + 1, 0)),
            pl.BlockSpec((1, DIM, HALO, W),
                         lambda i: (0, 0, (TH // HALO) * i + TH // HALO - 1, 0)),
            pl.BlockSpec((1, DIM, HALO, W),
                         lambda i: (0, 0, (TH // HALO) * (i + 2), 0)),
            pl.BlockSpec((1, DIM, TH, W), lambda i: (0, 0, i, 0)),
            full((E, DIM)), full((E, 1)),
            full((ER, DIM)), full((ER, DIM)),
            full((E, 3, RANK, 3 * RANK)), full((E, RANK, 1, 1)),
            full((E, 7, 2 * RANK, 7 * RANK)), full((E, 2 * RANK, 1, 1)),
            full((E, RANK, 1, 1)), full((E, RANK, 1, 1)),
            full((ER, ER)), full((ER, 1)),
            full((DIM, ER)), full((DIM, DIM)), full((DIM, 1)),
            full((64, 128)), full((128, 64)),
        ],
        out_specs=pl.BlockSpec((1, DIM, TH, W), lambda i: (0, 0, i, 0)),
    )
    y = pl.pallas_call(
        _strip_kernel,
        grid_spec=grid_spec,
        out_shape=jax.ShapeDtypeStruct((1, DIM, H, W), jnp.float32),
        compiler_params=pltpu.CompilerParams(
            dimension_semantics=("arbitrary",),
            vmem_limit_bytes=128 * 1024 * 1024,
        ),
    )(xpad, xpad, xpad, shared, router_w, rb, p0cat, p1cat, m3, qdb, m7,
      kvdb, lnw, lnb, powbd, pob, w2, projout_w, ob,
      jnp.asarray(_FWD), jnp.asarray(_INV))
    return y


# speedup vs baseline: 1.0959x; 1.0959x over previous
import numpy as np
import jax
import jax.numpy as jnp
from jax.experimental import pallas as pl
from jax.experimental.pallas import tpu as pltpu

DIM = 96
RANK = 16
E = 4
PS = 8
H = 384
W = 384
TH = 8
HALO = 8
NSTRIP = H // TH
ER = E * RANK

_m = np.arange(PS)
_ang = 2.0 * np.pi * np.outer(_m, _m) / PS
_C = np.cos(_ang)
_S = -np.sin(_ang)
_CR = (np.kron(_C, _C) - np.kron(_S, _S)).astype(np.float32)
_CI = (np.kron(_C, _S) + np.kron(_S, _C)).astype(np.float32)
_FWD = np.concatenate([_CR, _CI], axis=1)
_INV = np.concatenate([_CR, _CI], axis=0)


def _shift_cat(t, pad):
    tp = jnp.pad(t, ((0, 0), (0, 0), (pad, pad)))
    return jnp.concatenate([tp[:, :, dx:dx + W] for dx in range(2 * pad + 1)],
                           axis=0)


def _strip_kernel(xc_ref, xa_ref, xb_ref, sh_ref,
                  rw_ref, rb_ref,
                  p0_ref, p1_ref,
                  m3_ref, qdb_ref, m7_ref, kvdb_ref,
                  lnw_ref, lnb_ref, pow_ref, pob_ref,
                  w2_ref, ow_ref, ob_ref, cr_ref, ci_ref,
                  out_ref):
    N = TH * W
    xc = xc_ref[0]
    X = jnp.concatenate([xa_ref[0], xc, xb_ref[0]], axis=1)
    Xs = X[:, HALO - 3:HALO + TH + 3]
    xflat = xc.reshape(DIM, N)
    CRm = cr_ref[...]
    CIm = ci_ref[...]

    logits = jnp.dot(rw_ref[...], xflat,
                     preferred_element_type=jnp.float32) + rb_ref[...]
    lm = jnp.max(logits, axis=0, keepdims=True)
    ex_ = jnp.exp(logits - lm)
    probs = ex_ / jnp.sum(ex_, axis=0, keepdims=True)
    mx = jnp.max(probs, axis=0)
    taken = jnp.zeros((N,), jnp.bool_)
    gates = []
    for i in range(E):
        hit = jnp.logical_and(probs[i] == mx, jnp.logical_not(taken))
        gates.append(jnp.where(hit, mx, 0.0))
        taken = jnp.logical_or(taken, hit)

    bf = jnp.bfloat16
    h_all = jnp.dot(p0_ref[...].astype(bf),
                    Xs.reshape(DIM, (TH + 6) * W).astype(bf),
                    preferred_element_type=jnp.float32)
    h_all = h_all.reshape(ER, TH + 6, W).astype(bf)
    g_all = jnp.dot(p1_ref[...].astype(bf),
                    sh_ref[0].reshape(DIM, N).astype(bf),
                    preferred_element_type=jnp.float32)
    g_all = g_all * jax.nn.sigmoid(g_all)

    nh, nw = TH // PS, W // PS
    B = ER * nh * nw

    def to_patch(t):
        t = t.reshape(ER, nh, PS, nw, PS).transpose(0, 1, 3, 2, 4)
        return t.reshape(B, PS * PS)

    def from_patch(t):
        t = t.reshape(ER, nh, nw, PS, PS).transpose(0, 1, 3, 2, 4)
        return t.reshape(ER, TH, W)

    qpieces, kpieces, vpieces = [], [], []
    for i in range(E):
        h = h_all[RANK * i:RANK * (i + 1)]
        H3 = _shift_cat(h[:, 2:TH + 4], 1)
        qq = qdb_ref[i].reshape(RANK, 1)
        for dy in range(3):
            qq = qq + jnp.dot(m3_ref[i, dy].astype(bf),
                              H3[:, dy:dy + TH].reshape(3 * RANK, N),
                              preferred_element_type=jnp.float32)
        H7 = _shift_cat(h, 3)
        kv = kvdb_ref[i].reshape(2 * RANK, 1)
        for dy in range(7):
            kv = kv + jnp.dot(m7_ref[i, dy].astype(bf),
                              H7[:, dy:dy + TH].reshape(7 * RANK, N),
                              preferred_element_type=jnp.float32)
        qpieces.append(qq.reshape(RANK, TH, W))
        kpieces.append(kv[:RANK].reshape(RANK, TH, W))
        vpieces.append(kv[RANK:].reshape(RANK, TH, W))

    qcat = jnp.concatenate(qpieces, axis=0)
    kcat = jnp.concatenate(kpieces, axis=0)
    vcat = jnp.concatenate(vpieces, axis=0)
    QK = jnp.concatenate([to_patch(qcat), to_patch(kcat)], axis=0)
    F = jnp.dot(QK.astype(bf), CRm.astype(bf),
                preferred_element_type=jnp.float32)
    Aq, Bq = F[:B, :64], F[:B, 64:]
    Ak, Bk = F[B:, :64], F[B:, 64:]
    ReIm = jnp.concatenate([Aq * Ak - Bq * Bk, Aq * Bk + Bq * Ak], axis=1)
    o = jnp.dot(ReIm.astype(bf), CIm.astype(bf),
                preferred_element_type=jnp.float32)
    o = from_patch(o * (1.0 / 64.0))
    og = o.reshape(E, RANK, TH, W)
    mu = jnp.mean(og, axis=1, keepdims=True)
    var = jnp.mean((og - mu) ** 2, axis=1, keepdims=True)
    og = (og - mu) * jax.lax.rsqrt(var + 1e-5)
    og = og * lnw_ref[...] + lnb_ref[...]
    o = og.reshape(ER, TH, W) * vcat
    a = jnp.dot(pow_ref[...].astype(bf), o.reshape(ER, N).astype(bf),
                preferred_element_type=jnp.float32) + pob_ref[...]
    grep = jnp.broadcast_to(
        jnp.stack(gates)[:, None, :], (E, RANK, N)).reshape(ER, N)
    U = a * g_all * grep
    y = jnp.dot(w2_ref[...].astype(bf), U.astype(bf),
                preferred_element_type=jnp.float32)
    y = y + jnp.dot(ow_ref[...].astype(bf), (xflat * mx[None, :]).astype(bf),
                    preferred_element_type=jnp.float32)
    y = y + ob_ref[...]
    out_ref[...] = y.reshape(1, DIM, TH, W)


def kernel(x, shared, router_w, router_b, e_p0, e_p1, e_p2, e_qw, e_qdw,
           e_qdb, e_kvw, e_kvdw, e_kvdb, e_lnw, e_lnb, e_pow, e_pob,
           projout_w, projout_b):
    xpad = jnp.pad(x, ((0, 0), (0, 0), (TH, TH), (0, 0)))
    m3 = jnp.einsum('eoyx,eoc->eyoxc', e_qdw[:, :, 0], e_qw)
    m3 = m3.reshape(E, 3, RANK, 3 * RANK)
    m7 = jnp.einsum('eoyx,eoc->eyoxc', e_kvdw[:, :, 0], e_kvw)
    m7 = m7.reshape(E, 7, 2 * RANK, 7 * RANK)
    p0cat = e_p0.reshape(ER, DIM)
    p1cat = e_p1.reshape(ER, DIM)
    w2 = projout_w @ jnp.moveaxis(e_p2, 0, 1).reshape(DIM, ER)
    powbd = jnp.einsum('eoc,ef->eofc', e_pow,
                       jnp.eye(E, dtype=jnp.float32)).reshape(ER, ER)
    pob = e_pob.reshape(ER, 1)
    qdb = e_qdb.reshape(E, RANK, 1, 1)
    kvdb = e_kvdb.reshape(E, 2 * RANK, 1, 1)
    lnw = e_lnw.reshape(E, RANK, 1, 1)
    lnb = e_lnb.reshape(E, RANK, 1, 1)
    rb = router_b.reshape(E, 1)
    ob = projout_b.reshape(DIM, 1)

    full = lambda s: pl.BlockSpec(s, lambda i: (0,) * len(s))
    grid_spec = pl.GridSpec(
        grid=(NSTRIP,),
        in_specs=[
            pl.BlockSpec((1, DIM, TH, W), lambda i: (0, 0, i + 1, 0)),
            pl.BlockSpec((1, DIM, HALO, W),
                         lambda i: (0, 0, (TH // HALO) * i + TH // HALO - 1, 0)),
            pl.BlockSpec((1, DIM, HALO, W),
                         lambda i: (0, 0, (TH // HALO) * (i + 2), 0)),
            pl.BlockSpec((1, DIM, TH, W), lambda i: (0, 0, i, 0)),
            full((E, DIM)), full((E, 1)),
            full((ER, DIM)), full((ER, DIM)),
            full((E, 3, RANK, 3 * RANK)), full((E, RANK, 1, 1)),
            full((E, 7, 2 * RANK, 7 * RANK)), full((E, 2 * RANK, 1, 1)),
            full((E, RANK, 1, 1)), full((E, RANK, 1, 1)),
            full((ER, ER)), full((ER, 1)),
            full((DIM, ER)), full((DIM, DIM)), full((DIM, 1)),
            full((64, 128)), full((128, 64)),
        ],
        out_specs=pl.BlockSpec((1, DIM, TH, W), lambda i: (0, 0, i, 0)),
    )
    y = pl.pallas_call(
        _strip_kernel,
        grid_spec=grid_spec,
        out_shape=jax.ShapeDtypeStruct((1, DIM, H, W), jnp.float32),
        compiler_params=pltpu.CompilerParams(
            dimension_semantics=("arbitrary",),
            vmem_limit_bytes=128 * 1024 * 1024,
        ),
    )(xpad, xpad, xpad, shared, router_w, rb, p0cat, p1cat, m3, qdb, m7,
      kvdb, lnw, lnb, powbd, pob, w2, projout_w, ob,
      jnp.asarray(_FWD), jnp.asarray(_INV))
    return y


# speedup vs baseline: 1.2621x; 1.1517x over previous
import numpy as np
import jax
import jax.numpy as jnp
from jax.experimental import pallas as pl
from jax.experimental.pallas import tpu as pltpu

DIM = 96
RANK = 16
E = 4
PS = 8
H = 384
W = 384
TH = 16
HALO = 8
NSTRIP = H // TH
ER = E * RANK

_m = np.arange(PS)
_ang = 2.0 * np.pi * np.outer(_m, _m) / PS
_C = np.cos(_ang)
_S = -np.sin(_ang)
_CR = (np.kron(_C, _C) - np.kron(_S, _S)).astype(np.float32)
_CI = (np.kron(_C, _S) + np.kron(_S, _C)).astype(np.float32)
_FWD = np.concatenate([_CR, _CI], axis=1)
_INV = np.concatenate([_CR, _CI], axis=0)


def _shift_cat(t, pad):
    tp = jnp.pad(t, ((0, 0), (0, 0), (pad, pad)))
    return jnp.concatenate([tp[:, :, dx:dx + W] for dx in range(2 * pad + 1)],
                           axis=0)


def _strip_kernel(xc_ref, xa_ref, xb_ref, sh_ref,
                  rw_ref, rb_ref,
                  p0_ref, p1_ref,
                  m3_ref, qdb_ref, m7_ref, kvdb_ref,
                  lnw_ref, lnb_ref, pow_ref, pob_ref,
                  w2_ref, ow_ref, ob_ref, cr_ref, ci_ref,
                  out_ref):
    N = TH * W
    xc = xc_ref[0]
    X = jnp.concatenate([xa_ref[0], xc, xb_ref[0]], axis=1)
    Xs = X[:, HALO - 3:HALO + TH + 3]
    xflat = xc.reshape(DIM, N)
    CRm = cr_ref[...]
    CIm = ci_ref[...]

    logits = jnp.dot(rw_ref[...], xflat,
                     preferred_element_type=jnp.float32) + rb_ref[...]
    lm = jnp.max(logits, axis=0, keepdims=True)
    ex_ = jnp.exp(logits - lm)
    probs = ex_ / jnp.sum(ex_, axis=0, keepdims=True)
    mx = jnp.max(probs, axis=0)
    taken = jnp.zeros((N,), jnp.bool_)
    gates = []
    for i in range(E):
        hit = jnp.logical_and(probs[i] == mx, jnp.logical_not(taken))
        gates.append(jnp.where(hit, mx, 0.0))
        taken = jnp.logical_or(taken, hit)

    bf = jnp.bfloat16
    h_all = jnp.dot(p0_ref[...].astype(bf),
                    Xs.reshape(DIM, (TH + 6) * W).astype(bf),
                    preferred_element_type=jnp.float32)
    h_all = h_all.reshape(ER, TH + 6, W).astype(bf)
    g_all = jnp.dot(p1_ref[...].astype(bf),
                    sh_ref[0].reshape(DIM, N).astype(bf),
                    preferred_element_type=jnp.float32)
    g_all = g_all * jax.nn.sigmoid(g_all)

    nh, nw = TH // PS, W // PS
    B = RANK * nh * nw

    def to_patch(t):
        t = t.reshape(RANK, nh, PS, nw, PS).transpose(0, 1, 3, 2, 4)
        return t.reshape(B, PS * PS)

    def from_patch(t):
        t = t.reshape(RANK, nh, nw, PS, PS).transpose(0, 1, 3, 2, 4)
        return t.reshape(RANK, TH, W)

    upieces = []
    for i in range(E):
        h = h_all[RANK * i:RANK * (i + 1)]
        H3 = _shift_cat(h[:, 2:TH + 4], 1)
        qq = qdb_ref[i].reshape(RANK, 1)
        for dy in range(3):
            qq = qq + jnp.dot(m3_ref[i, dy].astype(bf),
                              H3[:, dy:dy + TH].reshape(3 * RANK, N),
                              preferred_element_type=jnp.float32)
        H7 = _shift_cat(h, 3)
        kv = kvdb_ref[i].reshape(2 * RANK, 1)
        for dy in range(7):
            kv = kv + jnp.dot(m7_ref[i, dy].astype(bf),
                              H7[:, dy:dy + TH].reshape(7 * RANK, N),
                              preferred_element_type=jnp.float32)
        kk = kv[:RANK].reshape(RANK, TH, W)
        vv = kv[RANK:].reshape(RANK, TH, W)
        QK = jnp.concatenate(
            [to_patch(qq.reshape(RANK, TH, W)), to_patch(kk)], axis=0)
        F = jnp.dot(QK.astype(bf), CRm.astype(bf),
                    preferred_element_type=jnp.float32)
        Aq, Bq = F[:B, :64], F[:B, 64:]
        Ak, Bk = F[B:, :64], F[B:, 64:]
        ReIm = jnp.concatenate(
            [Aq * Ak - Bq * Bk, Aq * Bk + Bq * Ak], axis=1)
        o = jnp.dot(ReIm.astype(bf), CIm.astype(bf),
                    preferred_element_type=jnp.float32)
        o = from_patch(o * (1.0 / 64.0))
        mu = jnp.mean(o, axis=0, keepdims=True)
        var = jnp.mean((o - mu) ** 2, axis=0, keepdims=True)
        o = (o - mu) * jax.lax.rsqrt(var + 1e-5)
        o = o * lnw_ref[i] + lnb_ref[i]
        o = o * vv
        a = jnp.dot(pow_ref[i].astype(bf), o.reshape(RANK, N).astype(bf),
                    preferred_element_type=jnp.float32)
        a = a + pob_ref[i].reshape(RANK, 1)
        u = a * g_all[RANK * i:RANK * (i + 1)] * gates[i][None, :]
        upieces.append(u)

    U = jnp.concatenate(upieces, axis=0)
    y = jnp.dot(w2_ref[...].astype(bf), U.astype(bf),
                preferred_element_type=jnp.float32)
    y = y + jnp.dot(ow_ref[...].astype(bf), (xflat * mx[None, :]).astype(bf),
                    preferred_element_type=jnp.float32)
    y = y + ob_ref[...]
    out_ref[...] = y.reshape(1, DIM, TH, W)


def kernel(x, shared, router_w, router_b, e_p0, e_p1, e_p2, e_qw, e_qdw,
           e_qdb, e_kvw, e_kvdw, e_kvdb, e_lnw, e_lnb, e_pow, e_pob,
           projout_w, projout_b):
    xpad = jnp.pad(x, ((0, 0), (0, 0), (TH, TH), (0, 0)))
    m3 = jnp.einsum('eoyx,eoc->eyoxc', e_qdw[:, :, 0], e_qw)
    m3 = m3.reshape(E, 3, RANK, 3 * RANK)
    m7 = jnp.einsum('eoyx,eoc->eyoxc', e_kvdw[:, :, 0], e_kvw)
    m7 = m7.reshape(E, 7, 2 * RANK, 7 * RANK)
    p0cat = e_p0.reshape(ER, DIM)
    p1cat = e_p1.reshape(ER, DIM)
    w2 = projout_w @ jnp.moveaxis(e_p2, 0, 1).reshape(DIM, ER)
    qdb = e_qdb.reshape(E, RANK, 1, 1)
    kvdb = e_kvdb.reshape(E, 2 * RANK, 1, 1)
    lnw = e_lnw.reshape(E, RANK, 1, 1)
    lnb = e_lnb.reshape(E, RANK, 1, 1)
    rb = router_b.reshape(E, 1)
    ob = projout_b.reshape(DIM, 1)

    full = lambda s: pl.BlockSpec(s, lambda i: (0,) * len(s))
    grid_spec = pl.GridSpec(
        grid=(NSTRIP,),
        in_specs=[
            pl.BlockSpec((1, DIM, TH, W), lambda i: (0, 0, i + 1, 0)),
            pl.BlockSpec((1, DIM, HALO, W),
                         lambda i: (0, 0, (TH // HALO) * i + TH // HALO - 1, 0)),
            pl.BlockSpec((1, DIM, HALO, W),
                         lambda i: (0, 0, (TH // HALO) * (i + 2), 0)),
            pl.BlockSpec((1, DIM, TH, W), lambda i: (0, 0, i, 0)),
            full((E, DIM)), full((E, 1)),
            full((ER, DIM)), full((ER, DIM)),
            full((E, 3, RANK, 3 * RANK)), full((E, RANK, 1, 1)),
            full((E, 7, 2 * RANK, 7 * RANK)), full((E, 2 * RANK, 1, 1)),
            full((E, RANK, 1, 1)), full((E, RANK, 1, 1)),
            full((E, RANK, RANK)), full((E, RANK)),
            full((DIM, ER)), full((DIM, DIM)), full((DIM, 1)),
            full((64, 128)), full((128, 64)),
        ],
        out_specs=pl.BlockSpec((1, DIM, TH, W), lambda i: (0, 0, i, 0)),
    )
    y = pl.pallas_call(
        _strip_kernel,
        grid_spec=grid_spec,
        out_shape=jax.ShapeDtypeStruct((1, DIM, H, W), jnp.float32),
        compiler_params=pltpu.CompilerParams(
            dimension_semantics=("arbitrary",),
            vmem_limit_bytes=128 * 1024 * 1024,
        ),
    )(xpad, xpad, xpad, shared, router_w, rb, p0cat, p1cat, m3, qdb, m7,
      kvdb, lnw, lnb, e_pow, e_pob, w2, projout_w, ob,
      jnp.asarray(_FWD), jnp.asarray(_INV))
    return y


# speedup vs baseline: 1.3126x; 1.0400x over previous
import numpy as np
import jax
import jax.numpy as jnp
from jax.experimental import pallas as pl
from jax.experimental.pallas import tpu as pltpu

DIM = 96
RANK = 16
E = 4
PS = 8
H = 384
W = 384
TH = 24
HALO = 8
NSTRIP = H // TH
ER = E * RANK

_m = np.arange(PS)
_ang = 2.0 * np.pi * np.outer(_m, _m) / PS
_C = np.cos(_ang)
_S = -np.sin(_ang)
_CR = (np.kron(_C, _C) - np.kron(_S, _S)).astype(np.float32)
_CI = (np.kron(_C, _S) + np.kron(_S, _C)).astype(np.float32)
_FWD = np.concatenate([_CR, _CI], axis=1)
_INV = np.concatenate([_CR, _CI], axis=0)


def _shift_cat(t, pad):
    tp = jnp.pad(t, ((0, 0), (0, 0), (pad, pad)))
    return jnp.concatenate([tp[:, :, dx:dx + W] for dx in range(2 * pad + 1)],
                           axis=0)


def _strip_kernel(xc_ref, xa_ref, xb_ref, sh_ref,
                  rw_ref, rb_ref,
                  p0_ref, p1_ref,
                  m3_ref, qdb_ref, m7_ref, kvdb_ref,
                  lnw_ref, lnb_ref, pow_ref, pob_ref,
                  w2_ref, ow_ref, ob_ref, cr_ref, ci_ref,
                  out_ref):
    N = TH * W
    xc = xc_ref[0]
    X = jnp.concatenate([xa_ref[0], xc, xb_ref[0]], axis=1)
    Xs = X[:, HALO - 3:HALO + TH + 3]
    xflat = xc.reshape(DIM, N)
    CRm = cr_ref[...]
    CIm = ci_ref[...]

    logits = jnp.dot(rw_ref[...], xflat,
                     preferred_element_type=jnp.float32) + rb_ref[...]
    lm = jnp.max(logits, axis=0, keepdims=True)
    ex_ = jnp.exp(logits - lm)
    probs = ex_ / jnp.sum(ex_, axis=0, keepdims=True)
    mx = jnp.max(probs, axis=0)
    taken = jnp.zeros((N,), jnp.bool_)
    gates = []
    for i in range(E):
        hit = jnp.logical_and(probs[i] == mx, jnp.logical_not(taken))
        gates.append(jnp.where(hit, mx, 0.0))
        taken = jnp.logical_or(taken, hit)

    bf = jnp.bfloat16
    h_all = jnp.dot(p0_ref[...].astype(bf),
                    Xs.reshape(DIM, (TH + 6) * W).astype(bf),
                    preferred_element_type=jnp.float32)
    h_all = h_all.reshape(ER, TH + 6, W).astype(bf)
    g_all = jnp.dot(p1_ref[...].astype(bf),
                    sh_ref[0].reshape(DIM, N).astype(bf),
                    preferred_element_type=jnp.float32)
    g_all = g_all * jax.nn.sigmoid(g_all)

    nh, nw = TH // PS, W // PS
    B = RANK * nh * nw

    def to_patch(t):
        t = t.reshape(RANK, nh, PS, nw, PS).transpose(0, 1, 3, 2, 4)
        return t.reshape(B, PS * PS)

    def from_patch(t):
        t = t.reshape(RANK, nh, nw, PS, PS).transpose(0, 1, 3, 2, 4)
        return t.reshape(RANK, TH, W)

    upieces = []
    for i in range(E):
        h = h_all[RANK * i:RANK * (i + 1)]
        H3 = _shift_cat(h[:, 2:TH + 4], 1)
        qq = qdb_ref[i].reshape(RANK, 1)
        for dy in range(3):
            qq = qq + jnp.dot(m3_ref[i, dy].astype(bf),
                              H3[:, dy:dy + TH].reshape(3 * RANK, N),
                              preferred_element_type=jnp.float32)
        H7 = _shift_cat(h, 3)
        kv = kvdb_ref[i].reshape(2 * RANK, 1)
        for dy in range(7):
            kv = kv + jnp.dot(m7_ref[i, dy].astype(bf),
                              H7[:, dy:dy + TH].reshape(7 * RANK, N),
                              preferred_element_type=jnp.float32)
        kk = kv[:RANK].reshape(RANK, TH, W)
        vv = kv[RANK:].reshape(RANK, TH, W)
        QK = jnp.concatenate(
            [to_patch(qq.reshape(RANK, TH, W)), to_patch(kk)], axis=0)
        F = jnp.dot(QK.astype(bf), CRm.astype(bf),
                    preferred_element_type=jnp.float32)
        Aq, Bq = F[:B, :64], F[:B, 64:]
        Ak, Bk = F[B:, :64], F[B:, 64:]
        ReIm = jnp.concatenate(
            [Aq * Ak - Bq * Bk, Aq * Bk + Bq * Ak], axis=1)
        o = jnp.dot(ReIm.astype(bf), CIm.astype(bf),
                    preferred_element_type=jnp.float32)
        o = from_patch(o * (1.0 / 64.0))
        mu = jnp.mean(o, axis=0, keepdims=True)
        var = jnp.mean((o - mu) ** 2, axis=0, keepdims=True)
        o = (o - mu) * jax.lax.rsqrt(var + 1e-5)
        o = o * lnw_ref[i] + lnb_ref[i]
        o = o * vv
        a = jnp.dot(pow_ref[i].astype(bf), o.reshape(RANK, N).astype(bf),
                    preferred_element_type=jnp.float32)
        a = a + pob_ref[i].reshape(RANK, 1)
        u = a * g_all[RANK * i:RANK * (i + 1)] * gates[i][None, :]
        upieces.append(u)

    U = jnp.concatenate(upieces, axis=0)
    y = jnp.dot(w2_ref[...].astype(bf), U.astype(bf),
                preferred_element_type=jnp.float32)
    y = y + jnp.dot(ow_ref[...].astype(bf), (xflat * mx[None, :]).astype(bf),
                    preferred_element_type=jnp.float32)
    y = y + ob_ref[...]
    out_ref[...] = y.reshape(1, DIM, TH, W)


def kernel(x, shared, router_w, router_b, e_p0, e_p1, e_p2, e_qw, e_qdw,
           e_qdb, e_kvw, e_kvdw, e_kvdb, e_lnw, e_lnb, e_pow, e_pob,
           projout_w, projout_b):
    xpad = jnp.pad(x, ((0, 0), (0, 0), (TH, TH), (0, 0)))
    m3 = jnp.einsum('eoyx,eoc->eyoxc', e_qdw[:, :, 0], e_qw)
    m3 = m3.reshape(E, 3, RANK, 3 * RANK)
    m7 = jnp.einsum('eoyx,eoc->eyoxc', e_kvdw[:, :, 0], e_kvw)
    m7 = m7.reshape(E, 7, 2 * RANK, 7 * RANK)
    p0cat = e_p0.reshape(ER, DIM)
    p1cat = e_p1.reshape(ER, DIM)
    w2 = projout_w @ jnp.moveaxis(e_p2, 0, 1).reshape(DIM, ER)
    qdb = e_qdb.reshape(E, RANK, 1, 1)
    kvdb = e_kvdb.reshape(E, 2 * RANK, 1, 1)
    lnw = e_lnw.reshape(E, RANK, 1, 1)
    lnb = e_lnb.reshape(E, RANK, 1, 1)
    rb = router_b.reshape(E, 1)
    ob = projout_b.reshape(DIM, 1)

    full = lambda s: pl.BlockSpec(s, lambda i: (0,) * len(s))
    grid_spec = pl.GridSpec(
        grid=(NSTRIP,),
        in_specs=[
            pl.BlockSpec((1, DIM, TH, W), lambda i: (0, 0, i + 1, 0)),
            pl.BlockSpec((1, DIM, HALO, W),
                         lambda i: (0, 0, (TH // HALO) * i + TH // HALO - 1, 0)),
            pl.BlockSpec((1, DIM, HALO, W),
                         lambda i: (0, 0, (TH // HALO) * (i + 2), 0)),
            pl.BlockSpec((1, DIM, TH, W), lambda i: (0, 0, i, 0)),
            full((E, DIM)), full((E, 1)),
            full((ER, DIM)), full((ER, DIM)),
            full((E, 3, RANK, 3 * RANK)), full((E, RANK, 1, 1)),
            full((E, 7, 2 * RANK, 7 * RANK)), full((E, 2 * RANK, 1, 1)),
            full((E, RANK, 1, 1)), full((E, RANK, 1, 1)),
            full((E, RANK, RANK)), full((E, RANK)),
            full((DIM, ER)), full((DIM, DIM)), full((DIM, 1)),
            full((64, 128)), full((128, 64)),
        ],
        out_specs=pl.BlockSpec((1, DIM, TH, W), lambda i: (0, 0, i, 0)),
    )
    y = pl.pallas_call(
        _strip_kernel,
        grid_spec=grid_spec,
        out_shape=jax.ShapeDtypeStruct((1, DIM, H, W), jnp.float32),
        compiler_params=pltpu.CompilerParams(
            dimension_semantics=("arbitrary",),
            vmem_limit_bytes=128 * 1024 * 1024,
        ),
    )(xpad, xpad, xpad, shared, router_w, rb, p0cat, p1cat, m3, qdb, m7,
      kvdb, lnw, lnb, e_pow, e_pob, w2, projout_w, ob,
      jnp.asarray(_FWD), jnp.asarray(_INV))
    return y
